# baseline (speedup 1.0000x reference)
import functools

import jax
import jax.numpy as jnp
from jax import lax
from jax.experimental import pallas as pl
from jax.experimental.pallas import tpu as pltpu
from jax.experimental.pallas import tpu_sc as plsc

_NUM_M = 495
_NUM_D = 383
_OFF = 383
_DIM = 128
_LAYERS = 3
_N_EDGE = 5430
_EB = 512

_MP = 512
_DP = 384
_NS_WORDS = _MP * _DP
_DUMP = _NS_WORDS - 1

_NTILE = 16
_EPT = 384
_EPAD = _NTILE * _EPT
_ROWS_PER_TILE = 3
_STRIPE = _NS_WORDS // _NTILE
_ZCHUNK = 1024

_RT2 = (((1,), (1,)), ((), ()))


def _mlp_body(m_sim_ref, d_sim_ref, Wm_ref, Wd_ref, A1W_ref,
              A1b_ref, A2W_ref, tm_ref, td_ref,
              idx_ref, val_ref, ed_ref):
    f32 = jnp.float32
    dg = lax.dot_general
    Em = dg(m_sim_ref[...], Wm_ref[...], _RT2, preferred_element_type=f32)
    Ed = dg(d_sim_ref[...], Wd_ref[...], _RT2, preferred_element_type=f32)
    EmT = Em.T
    EdT = Ed.T

    A1m = A1W_ref[:, :_DIM]
    A1d = A1W_ref[:, _DIM:]
    A1b_col = A1b_ref[...].reshape(1, _DIM).T
    A2 = A2W_ref[...]

    tail0 = (_N_EDGE // _EB) * _EB
    ntail = _EPAD - tail0
    idx_ref[pl.ds(tail0, ntail)] = jnp.full((1, ntail), _DUMP, jnp.int32).reshape(ntail)
    val_ref[pl.ds(tail0, ntail)] = jnp.zeros((1, ntail), f32).reshape(ntail)

    for start in range(0, _N_EDGE, _EB):
        nb = min(_EB, _N_EDGE - start)
        tm_r = tm_ref[start:start + nb].reshape(1, nb)
        td_r = td_ref[start:start + nb].reshape(1, nb)
        oh_mT = (tm_r == _OFF + lax.broadcasted_iota(jnp.int32, (_NUM_M, nb), 0)
                 ).astype(f32)
        oh_dT = (td_r == lax.broadcasted_iota(jnp.int32, (_NUM_D, nb), 0)
                 ).astype(f32)
        meT = jnp.dot(EmT, oh_mT, preferred_element_type=f32)
        deT = jnp.dot(EdT, oh_dT, preferred_element_type=f32)
        hT = jnp.tanh(jnp.dot(A1m, jnp.maximum(meT, 0.0), preferred_element_type=f32)
                      + jnp.dot(A1d, jnp.maximum(deT, 0.0), preferred_element_type=f32)
                      + A1b_col)
        sc = jnp.dot(A2, hT, preferred_element_type=f32)
        vals = jnp.exp(sc)
        flat = (tm_r - _OFF) * _DP + td_r
        idx_ref[pl.ds(start, nb)] = flat.reshape(nb)
        val_ref[pl.ds(start, nb)] = vals.reshape(nb)
    ed_ref[...] = Ed


def _scatter_body(idx_hbm, val_hbm, s_hbm, idx_v, val_v, zbuf, s_shared):
    cid = lax.axis_index("c")
    sid = lax.axis_index("s")

    @pl.when(cid == 0)
    def _():
        pltpu.sync_copy(idx_hbm, idx_v)
        pltpu.sync_copy(val_hbm, val_v)
        for k in range(_ZCHUNK // 16):
            zbuf[pl.ds(k * 16, 16)] = jnp.zeros((16,), jnp.float32)
        for k in range(_STRIPE // _ZCHUNK):
            pltpu.sync_copy(zbuf, s_shared.at[pl.ds(sid * _STRIPE + k * _ZCHUNK, _ZCHUNK)])
        plsc.subcore_barrier()
        for j in range(_ROWS_PER_TILE):
            row = sid * _ROWS_PER_TILE + j
            pltpu.sync_copy(val_v.at[row], s_shared.at[idx_v.at[row]], add=True)
        plsc.subcore_barrier()
        pltpu.sync_copy(s_shared.at[pl.ds(sid * _STRIPE, _STRIPE)],
                        s_hbm.at[pl.ds(sid * _STRIPE, _STRIPE)])


def _prop_body(s_ref, ed_ref, out_m_ref, out_d_ref):
    f32 = jnp.float32
    S = s_ref[...][:_NUM_M, :_NUM_D]
    Ed = ed_ref[...]
    rowsum = jnp.sum(S, axis=1, keepdims=True)
    rm = jnp.where(rowsum > 0.0, 1.0 / rowsum, 0.0)
    ST = S.T
    colsum = jnp.sum(ST, axis=1, keepdims=True)
    rd = jnp.where(colsum > 0.0, 1.0 / colsum, 0.0)

    m_acc = jnp.zeros((_NUM_M, _DIM), f32)
    d_acc = jnp.zeros((_NUM_D, _DIM), f32)
    d_emb = Ed
    for _ in range(_LAYERS):
        m_emb = jnp.tanh(jnp.dot(S, d_emb, preferred_element_type=f32) * rm)
        d_emb = jnp.tanh(jnp.dot(ST, m_emb, preferred_element_type=f32) * rd)
        m_acc = m_acc + m_emb
        d_acc = d_acc + d_emb
    out_m_ref[...] = m_acc
    out_d_ref[...] = d_acc


def kernel(m_sim, d_sim, W_m, W_d, A1_W, A1_b, A2_W, train_mirna, train_disease):
    f32 = jnp.float32
    idx, vals, Ed = pl.pallas_call(
        _mlp_body,
        out_shape=(
            jax.ShapeDtypeStruct((_EPAD,), jnp.int32),
            jax.ShapeDtypeStruct((_EPAD,), f32),
            jax.ShapeDtypeStruct((_NUM_D, _DIM), f32),
        ),
    )(m_sim, d_sim, W_m, W_d, A1_W, A1_b, A2_W, train_mirna, train_disease)

    mesh = plsc.VectorSubcoreMesh(core_axis_name="c", subcore_axis_name="s",
                                  num_cores=2, num_subcores=16)
    s_flat = pl.kernel(
        _scatter_body,
        out_type=jax.ShapeDtypeStruct((_NS_WORDS,), f32),
        mesh=mesh,
        scratch_types=[
            pltpu.VMEM((_EPAD // 128, 128), jnp.int32),
            pltpu.VMEM((_EPAD // 128, 128), f32),
            pltpu.VMEM((_ZCHUNK,), f32),
            pltpu.VMEM_SHARED((_NS_WORDS,), f32),
        ],
    )(idx.reshape(_EPAD // 128, 128), vals.reshape(_EPAD // 128, 128))

    out_m, out_d = pl.pallas_call(
        _prop_body,
        out_shape=(
            jax.ShapeDtypeStruct((_NUM_M, _DIM), f32),
            jax.ShapeDtypeStruct((_NUM_D, _DIM), f32),
        ),
    )(s_flat.reshape(_MP, _DP), Ed)
    return (out_m, out_d)


# speedup vs baseline: 2.6565x; 2.6565x over previous
import jax
import jax.numpy as jnp
from jax.experimental import pallas as pl
from jax.experimental.pallas import tpu as pltpu

_NUM_M = 495
_NUM_D = 383
_OFF = 383
_DIM = 128
_LAYERS = 3
_N_EDGE = 5430
_EB = 512

_RT2 = (((1,), (1,)), ((), ()))


def _body(m_sim_ref, d_sim_ref, Wm_ref, Wd_ref, A1W_ref,
          A1b_ref, A2W_ref, tm_ref, td_ref,
          out_m_ref, out_d_ref):
    f32 = jnp.float32
    dg = jax.lax.dot_general
    Em = dg(m_sim_ref[...], Wm_ref[...], _RT2, preferred_element_type=f32)
    Ed = dg(d_sim_ref[...], Wd_ref[...], _RT2, preferred_element_type=f32)
    EmT = Em.T
    EdT = Ed.T

    A1m = A1W_ref[:, :_DIM]
    A1d = A1W_ref[:, _DIM:]
    A1b_col = A1b_ref[...].reshape(1, _DIM).T
    A2 = A2W_ref[...]

    Pm = jnp.dot(A1m, jnp.maximum(EmT, 0.0), preferred_element_type=f32)
    Pd = jnp.dot(A1d, jnp.maximum(EdT, 0.0), preferred_element_type=f32) + A1b_col

    S = jnp.zeros((_NUM_M, _NUM_D), f32)
    for start in range(0, _N_EDGE, _EB):
        nb = min(_EB, _N_EDGE - start)
        tm_r = tm_ref[start:start + nb].reshape(1, nb)
        td_r = td_ref[start:start + nb].reshape(1, nb)
        cmp_m = tm_r == _OFF + jax.lax.broadcasted_iota(jnp.int32, (_NUM_M, nb), 0)
        cmp_d = td_r == jax.lax.broadcasted_iota(jnp.int32, (_NUM_D, nb), 0)
        oh_mT = cmp_m.astype(f32)
        oh_dT = cmp_d.astype(f32)
        hT = jnp.tanh(jnp.dot(Pm, oh_mT, preferred_element_type=f32)
                      + jnp.dot(Pd, oh_dT, preferred_element_type=f32))
        sc = jnp.dot(A2, hT, preferred_element_type=f32)
        vals = jnp.exp(sc)
        oh_dT_s = jnp.where(cmp_d, vals, 0.0)
        S = S + dg(oh_mT, oh_dT_s, _RT2, preferred_element_type=f32)

    rowsum = jnp.sum(S, axis=1, keepdims=True)
    rm = jnp.where(rowsum > 0.0, 1.0 / rowsum, 0.0)
    ST = S.T
    colsum = jnp.sum(ST, axis=1, keepdims=True)
    rd = jnp.where(colsum > 0.0, 1.0 / colsum, 0.0)

    m_acc = jnp.zeros((_NUM_M, _DIM), f32)
    d_acc = jnp.zeros((_NUM_D, _DIM), f32)
    d_emb = Ed
    for _ in range(_LAYERS):
        m_emb = jnp.tanh(jnp.dot(S, d_emb, preferred_element_type=f32) * rm)
        d_emb = jnp.tanh(jnp.dot(ST, m_emb, preferred_element_type=f32) * rd)
        m_acc = m_acc + m_emb
        d_acc = d_acc + d_emb
    out_m_ref[...] = m_acc
    out_d_ref[...] = d_acc


def kernel(m_sim, d_sim, W_m, W_d, A1_W, A1_b, A2_W, train_mirna, train_disease):
    f32 = jnp.float32
    return pl.pallas_call(
        _body,
        out_shape=(
            jax.ShapeDtypeStruct((_NUM_M, _DIM), f32),
            jax.ShapeDtypeStruct((_NUM_D, _DIM), f32),
        ),
    )(m_sim, d_sim, W_m, W_d, A1_W, A1_b, A2_W, train_mirna, train_disease)


# speedup vs baseline: 3.2302x; 1.2160x over previous
import jax
import jax.numpy as jnp
from jax.experimental import pallas as pl
from jax.experimental.pallas import tpu as pltpu

_NUM_M = 495
_NUM_D = 383
_OFF = 383
_DIM = 128
_LAYERS = 3
_N_EDGE = 5430
_EB = 5430

_RT2 = (((1,), (1,)), ((), ()))


def _body(m_sim_ref, d_sim_ref, Wm_ref, Wd_ref, A1W_ref,
          A1b_ref, A2W_ref, tm_ref, td_ref,
          out_m_ref, out_d_ref):
    f32 = jnp.float32
    dg = jax.lax.dot_general
    Em = dg(m_sim_ref[...], Wm_ref[...], _RT2, preferred_element_type=f32)
    Ed = dg(d_sim_ref[...], Wd_ref[...], _RT2, preferred_element_type=f32)
    EmT = Em.T
    EdT = Ed.T

    A1m = A1W_ref[:, :_DIM]
    A1d = A1W_ref[:, _DIM:]
    A1b_col = A1b_ref[...].reshape(1, _DIM).T
    A2 = A2W_ref[...]

    Pm = jnp.dot(A1m, jnp.maximum(EmT, 0.0), preferred_element_type=f32)
    Pd = jnp.dot(A1d, jnp.maximum(EdT, 0.0), preferred_element_type=f32) + A1b_col

    S = jnp.zeros((_NUM_M, _NUM_D), f32)
    for start in range(0, _N_EDGE, _EB):
        nb = min(_EB, _N_EDGE - start)
        tm_r = tm_ref[start:start + nb].reshape(1, nb)
        td_r = td_ref[start:start + nb].reshape(1, nb)
        cmp_m = tm_r == _OFF + jax.lax.broadcasted_iota(jnp.int32, (_NUM_M, nb), 0)
        cmp_d = td_r == jax.lax.broadcasted_iota(jnp.int32, (_NUM_D, nb), 0)
        oh_mT = cmp_m.astype(f32)
        oh_dT = cmp_d.astype(f32)
        hT = jnp.tanh(jnp.dot(Pm, oh_mT, preferred_element_type=f32)
                      + jnp.dot(Pd, oh_dT, preferred_element_type=f32))
        sc = jnp.dot(A2, hT, preferred_element_type=f32)
        vals = jnp.exp(sc)
        oh_dT_s = jnp.where(cmp_d, vals, 0.0)
        S = S + dg(oh_mT, oh_dT_s, _RT2, preferred_element_type=f32)

    rowsum = jnp.sum(S, axis=1, keepdims=True)
    rm = jnp.where(rowsum > 0.0, 1.0 / rowsum, 0.0)
    ST = S.T
    colsum = jnp.sum(ST, axis=1, keepdims=True)
    rd = jnp.where(colsum > 0.0, 1.0 / colsum, 0.0)

    m_acc = jnp.zeros((_NUM_M, _DIM), f32)
    d_acc = jnp.zeros((_NUM_D, _DIM), f32)
    d_emb = Ed
    for _ in range(_LAYERS):
        m_emb = jnp.tanh(jnp.dot(S, d_emb, preferred_element_type=f32) * rm)
        d_emb = jnp.tanh(jnp.dot(ST, m_emb, preferred_element_type=f32) * rd)
        m_acc = m_acc + m_emb
        d_acc = d_acc + d_emb
    out_m_ref[...] = m_acc
    out_d_ref[...] = d_acc


def kernel(m_sim, d_sim, W_m, W_d, A1_W, A1_b, A2_W, train_mirna, train_disease):
    f32 = jnp.float32
    return pl.pallas_call(
        _body,
        out_shape=(
            jax.ShapeDtypeStruct((_NUM_M, _DIM), f32),
            jax.ShapeDtypeStruct((_NUM_D, _DIM), f32),
        ),
    )(m_sim, d_sim, W_m, W_d, A1_W, A1_b, A2_W, train_mirna, train_disease)
